# baseline (device time: 68095 ns/iter reference)
import jax
import jax.numpy as jnp
from jax import lax
from jax.experimental import pallas as pl
from jax.experimental.pallas import tpu as pltpu

N_DEV = 32
LOG2_N = 5
B, Sq, D = 2, 256, 768
Hq, Dh = 8, 64
HD = Hq * Dh
BH = B * Hq
S = 4
SW = Sq // S
FH = Sq // 2


def kernel(x, Wq, Wo, K_ext, V_ext):
    skv_loc = K_ext.shape[1]

    def body(x_ref, wq_ref, wo_ref, k_ref, v_ref, out_ref,
             o_acc, ml_acc, o_rx, ml_rx,
             o_send_sems, o_recv_sems, ml_send_sems, ml_recv_sems):
        my = lax.axis_index("i")

        x2 = x_ref[...].reshape(B * Sq, D).astype(jnp.bfloat16)
        wq = wq_ref[...].astype(jnp.bfloat16)
        qT = lax.dot_general(wq, x2, (((0,), (1,)), ((), ())),
                             preferred_element_type=jnp.float32)
        qT = qT * 0.125

        kbs = []
        vbs = []
        for b in range(B):
            kbs.append(k_ref[b, :, :, :].reshape(skv_loc, HD).astype(jnp.bfloat16))
            vbs.append(v_ref[b, :, :, :].reshape(skv_loc, HD).astype(jnp.bfloat16))

        def flash_round(half):
            for b in range(B):
                for h in range(Hq):
                    bh = b * Hq + h
                    c0 = b * Sq + half * FH
                    q_bh = qT[h * Dh:(h + 1) * Dh,
                              c0:c0 + FH].astype(jnp.bfloat16)
                    k_bh = kbs[b][:, h * Dh:(h + 1) * Dh]
                    v_bh = vbs[b][:, h * Dh:(h + 1) * Dh]
                    sT = lax.dot_general(k_bh, q_bh, (((1,), (0,)), ((), ())),
                                         preferred_element_type=jnp.float32)
                    m = jnp.max(sT, axis=0, keepdims=True)
                    p = jnp.exp(sT - m)
                    l = jnp.sum(p, axis=0, keepdims=True)
                    oT = lax.dot_general(v_bh, p.astype(jnp.bfloat16),
                                         (((0,), (0,)), ((), ())),
                                         preferred_element_type=jnp.float32)
                    ob = oT.astype(jnp.bfloat16)
                    for q in range(2):
                        j = 2 * half + q
                        o_acc[j, bh, :, :] = ob[:, q * SW:(q + 1) * SW]
                        ml_acc[j, bh, 0:1, :] = m[:, q * SW:(q + 1) * SW]
                        ml_acc[j, bh, 1:2, :] = l[:, q * SW:(q + 1) * SW]

        def make_rdmas(j, step):
            bit = (step + j) % LOG2_N
            partner = my ^ (1 << bit) if isinstance(bit, int) else (
                my ^ jnp.left_shift(jnp.int32(1), bit.astype(jnp.int32)))
            o_rd = pltpu.make_async_remote_copy(
                src_ref=o_acc.at[j],
                dst_ref=o_rx.at[step, j],
                send_sem=o_send_sems.at[step, j],
                recv_sem=o_recv_sems.at[step, j],
                device_id=(partner,),
                device_id_type=pl.DeviceIdType.MESH,
            )
            ml_rd = pltpu.make_async_remote_copy(
                src_ref=ml_acc.at[j],
                dst_ref=ml_rx.at[step, j],
                send_sem=ml_send_sems.at[step, j],
                recv_sem=ml_recv_sems.at[step, j],
                device_id=(partner,),
                device_id_type=pl.DeviceIdType.MESH,
            )
            return o_rd, ml_rd

        def start_exchange(j, step):
            o_rd, ml_rd = make_rdmas(j, step)
            o_rd.start()
            ml_rd.start()

        def combine(j, step):
            m1 = ml_acc[j, :, 0:1, :]
            l1 = ml_acc[j, :, 1:2, :]
            m2 = ml_rx[step, j, :, 0:1, :]
            l2 = ml_rx[step, j, :, 1:2, :]
            mn = jnp.maximum(m1, m2)
            a1 = jnp.exp(m1 - mn)
            a2 = jnp.exp(m2 - mn)
            ml_acc[j, :, 0:1, :] = mn
            ml_acc[j, :, 1:2, :] = a1 * l1 + a2 * l2
            o_new = (a1 * o_acc[j].astype(jnp.float32)
                     + a2 * o_rx[step, j].astype(jnp.float32))
            o_acc[j] = o_new.astype(jnp.bfloat16)

        wo = wo_ref[...].astype(jnp.bfloat16)

        def project(j):
            half, q = divmod(j, 2)
            linv = 1.0 / ml_acc[j, :, 1:2, :]
            r0 = half * FH + q * SW
            for b in range(B):
                acc = jnp.zeros((SW, D), jnp.float32)
                for h in range(Hq):
                    bh = b * Hq + h
                    o_n = (o_acc[j, bh, :, :].astype(jnp.float32)
                           * linv[bh, :, :]).astype(jnp.bfloat16)
                    wo_h = wo[h * Dh:(h + 1) * Dh, :]
                    acc = acc + lax.dot_general(
                        o_n, wo_h, (((0,), (0,)), ((), ())),
                        preferred_element_type=jnp.float32)
                out_ref[b, r0:r0 + SW, :] = acc

        flash_round(0)
        start_exchange(0, 0)
        start_exchange(1, 0)
        flash_round(1)
        start_exchange(2, 0)
        start_exchange(3, 0)

        def step_body(step, carry):
            for j in range(S):
                o_rd, ml_rd = make_rdmas(j, step)
                o_rd.wait()
                ml_rd.wait()
                combine(j, step)

                @pl.when(step < LOG2_N - 1)
                def _():
                    start_exchange(j, jnp.minimum(step + 1, LOG2_N - 1))

                @pl.when(step == LOG2_N - 1)
                def _():
                    project(j)
            return carry

        lax.fori_loop(0, LOG2_N, step_body, 0)

    return pl.pallas_call(
        body,
        out_shape=jax.ShapeDtypeStruct((B, Sq, D), jnp.float32),
        in_specs=[pl.BlockSpec(memory_space=pltpu.VMEM)] * 5,
        out_specs=pl.BlockSpec(memory_space=pltpu.VMEM),
        scratch_shapes=[
            pltpu.VMEM((S, BH, Dh, SW), jnp.bfloat16),
            pltpu.VMEM((S, BH, 2, SW), jnp.float32),
            pltpu.VMEM((LOG2_N, S, BH, Dh, SW), jnp.bfloat16),
            pltpu.VMEM((LOG2_N, S, BH, 2, SW), jnp.float32),
            pltpu.SemaphoreType.DMA((LOG2_N, S)),
            pltpu.SemaphoreType.DMA((LOG2_N, S)),
            pltpu.SemaphoreType.DMA((LOG2_N, S)),
            pltpu.SemaphoreType.DMA((LOG2_N, S)),
        ],
    )(x, Wq, Wo, K_ext, V_ext)
